# baseline (speedup 1.0000x reference)
import jax
import jax.numpy as jnp
from jax.experimental import pallas as pl
from jax.experimental.pallas import tpu as pltpu

N = 4096
HID = 64
O = 128
THRESH = 0.6
BLK = 128
NP = N // BLK


def _u_kernel(f1_ref, f2_ref, w11_ref, w21_ref, u1_ref, u2_ref):
    u1_ref[...] = jnp.dot(f1_ref[...], w11_ref[...],
                          preferred_element_type=jnp.float32
                          ).astype(jnp.bfloat16)
    u2_ref[...] = jnp.dot(f2_ref[...], w21_ref[...],
                          preferred_element_type=jnp.float32
                          ).astype(jnp.bfloat16)


def _u(feat1, feat2, e1w1, e2w1):
    d1 = feat1.shape[1]
    d2 = feat2.shape[1]
    blk = 512
    return pl.pallas_call(
        _u_kernel,
        grid=(N // blk,),
        in_specs=[
            pl.BlockSpec((blk, d1), lambda i: (i, 0)),
            pl.BlockSpec((blk, d2), lambda i: (i, 0)),
            pl.BlockSpec((d1, HID), lambda i: (0, 0)),
            pl.BlockSpec((d2, HID), lambda i: (0, 0)),
        ],
        out_specs=[
            pl.BlockSpec((blk, HID), lambda i: (i, 0)),
            pl.BlockSpec((blk, HID), lambda i: (i, 0)),
        ],
        out_shape=[
            jax.ShapeDtypeStruct((N, HID), jnp.bfloat16),
            jax.ShapeDtypeStruct((N, HID), jnp.bfloat16),
        ],
    )(feat1, feat2, e1w1, e2w1)


def _encode(ab, u, w2, hn_ref, yin_ref, row):
    h = jnp.dot(ab, u, preferred_element_type=jnp.float32)
    h = jnp.maximum(h, 0.0)
    norm = jnp.sqrt(jnp.sum(h * h, axis=1, keepdims=True))
    hn_ref[pl.ds(row, BLK), :] = (h / jnp.maximum(norm, 1e-12)).astype(
        jnp.bfloat16)
    yin_ref[pl.ds(row, BLK), :] = jnp.dot(
        h, w2, preferred_element_type=jnp.float32).astype(jnp.bfloat16)


def _simagg(hn_ref, yin_ref, row):
    hnb = hn_ref[pl.ds(row, BLK), :]
    s = jax.lax.dot_general(
        hnb, hn_ref[...],
        dimension_numbers=(((1,), (1,)), ((), ())),
        preferred_element_type=jnp.float32)
    s = jnp.where(s < THRESH, 0.0, s)
    rs = jnp.sum(s, axis=1, keepdims=True)
    agg = jnp.dot(s.astype(jnp.bfloat16), yin_ref[...],
                  preferred_element_type=jnp.float32)
    return agg / jnp.maximum(rs, 1e-12)


def _mega_kernel(a1_ref, a2_ref, u1_ref, u2_ref, w12_ref, w22_ref,
                 dw1_ref, dw2_ref,
                 y1_ref, y2_ref, z_ref, r1_ref, r2_ref,
                 adjbf2_ref, hn1_ref, yin1_ref, hn2_ref, yin2_ref,
                 x1_ref, x2_ref, y1s_ref):
    i = pl.program_id(0)

    @pl.when(i < NP)
    def _p0():
        _encode(a1_ref[...].astype(jnp.bfloat16), u1_ref[...], w12_ref[...],
                hn1_ref, yin1_ref, i * BLK)

    @pl.when(jnp.logical_and(i >= NP, i < 2 * NP))
    def _p1():
        j = i - NP
        a2b = a2_ref[...].astype(jnp.bfloat16)
        _encode(a2b, u2_ref[...], w22_ref[...], hn2_ref, yin2_ref, j * BLK)
        adjbf2_ref[pl.ds(j * BLK, BLK), :] = a2b
        y = _simagg(hn1_ref, yin1_ref, j * BLK)
        y1_ref[...] = y
        y1s_ref[pl.ds(j * BLK, BLK), :] = y.astype(jnp.bfloat16)
        x1_ref[pl.ds(j * BLK, BLK), :] = jnp.dot(
            y, dw1_ref[...], preferred_element_type=jnp.float32
        ).astype(jnp.bfloat16)

    @pl.when(jnp.logical_and(i >= 2 * NP, i < 3 * NP))
    def _p2():
        k = i - 2 * NP
        y = _simagg(hn2_ref, yin2_ref, k * BLK)
        y2_ref[...] = y
        z_ref[...] = (y + y1s_ref[pl.ds(k * BLK, BLK), :].astype(
            jnp.float32)) * 0.5
        x2_ref[pl.ds(k * BLK, BLK), :] = jnp.dot(
            y, dw2_ref[...], preferred_element_type=jnp.float32
        ).astype(jnp.bfloat16)
        r1_ref[...] = jnp.dot(
            a1_ref[...].astype(jnp.bfloat16), x1_ref[...],
            preferred_element_type=jnp.float32)

    @pl.when(i >= 3 * NP)
    def _p3():
        m = i - 3 * NP
        r2_ref[...] = jnp.dot(
            adjbf2_ref[pl.ds(m * BLK, BLK), :], x2_ref[...],
            preferred_element_type=jnp.float32)


def _mega(adj1, adj2, u1, u2, e1w2, e2w2, d1w, d2w):
    d1 = d1w.shape[1]
    d2 = d2w.shape[1]
    return pl.pallas_call(
        _mega_kernel,
        grid=(4 * NP,),
        in_specs=[
            pl.BlockSpec(
                (BLK, N),
                lambda i: (jnp.where(i < NP, i,
                                     jnp.clip(i - 2 * NP, 0, NP - 1)), 0)),
            pl.BlockSpec(
                (BLK, N), lambda i: (jnp.clip(i - NP, 0, NP - 1), 0)),
            pl.BlockSpec((N, HID), lambda i: (0, 0)),
            pl.BlockSpec((N, HID), lambda i: (0, 0)),
            pl.BlockSpec((HID, O), lambda i: (0, 0)),
            pl.BlockSpec((HID, O), lambda i: (0, 0)),
            pl.BlockSpec((O, d1), lambda i: (0, 0)),
            pl.BlockSpec((O, d2), lambda i: (0, 0)),
        ],
        out_specs=[
            pl.BlockSpec((BLK, O), lambda i: (jnp.clip(i - NP, 0, NP - 1), 0)),
            pl.BlockSpec((BLK, O),
                         lambda i: (jnp.clip(i - 2 * NP, 0, NP - 1), 0)),
            pl.BlockSpec((BLK, O),
                         lambda i: (jnp.clip(i - 2 * NP, 0, NP - 1), 0)),
            pl.BlockSpec((BLK, d1),
                         lambda i: (jnp.clip(i - 2 * NP, 0, NP - 1), 0)),
            pl.BlockSpec((BLK, d2),
                         lambda i: (jnp.clip(i - 3 * NP, 0, NP - 1), 0)),
        ],
        out_shape=[
            jax.ShapeDtypeStruct((N, O), jnp.float32),
            jax.ShapeDtypeStruct((N, O), jnp.float32),
            jax.ShapeDtypeStruct((N, O), jnp.float32),
            jax.ShapeDtypeStruct((N, d1), jnp.float32),
            jax.ShapeDtypeStruct((N, d2), jnp.float32),
        ],
        scratch_shapes=[
            pltpu.VMEM((N, N), jnp.bfloat16),
            pltpu.VMEM((N, HID), jnp.bfloat16),
            pltpu.VMEM((N, O), jnp.bfloat16),
            pltpu.VMEM((N, HID), jnp.bfloat16),
            pltpu.VMEM((N, O), jnp.bfloat16),
            pltpu.VMEM((N, d1), jnp.bfloat16),
            pltpu.VMEM((N, d2), jnp.bfloat16),
            pltpu.VMEM((N, O), jnp.bfloat16),
        ],
    )(adj1, adj2, u1, u2, e1w2, e2w2, d1w, d2w)


def kernel(feat1, feat2, adj_spatial1, adj_spatial2,
           e1w1, e1w2, e2w1, e2w2, d1w, d2w):
    u1, u2 = _u(feat1, feat2, e1w1, e2w1)
    y1, y2, z, recon1, recon2 = _mega(adj_spatial1, adj_spatial2, u1, u2,
                                      e1w2, e2w2, d1w, d2w)
    return (y1, y2, z, recon1, recon2)


# speedup vs baseline: 1.1911x; 1.1911x over previous
import jax
import jax.numpy as jnp
from jax.experimental import pallas as pl
from jax.experimental.pallas import tpu as pltpu

N = 4096
HID = 64
O = 128
THRESH = 0.6
BLK = 256
NP = N // BLK


def _u_kernel(f1_ref, f2_ref, w11_ref, w21_ref, u1_ref, u2_ref):
    u1_ref[...] = jnp.dot(f1_ref[...], w11_ref[...],
                          preferred_element_type=jnp.float32)
    u2_ref[...] = jnp.dot(f2_ref[...], w21_ref[...],
                          preferred_element_type=jnp.float32)


def _u(feat1, feat2, e1w1, e2w1):
    d1 = feat1.shape[1]
    d2 = feat2.shape[1]
    blk = 512
    return pl.pallas_call(
        _u_kernel,
        grid=(N // blk,),
        in_specs=[
            pl.BlockSpec((blk, d1), lambda i: (i, 0)),
            pl.BlockSpec((blk, d2), lambda i: (i, 0)),
            pl.BlockSpec((d1, HID), lambda i: (0, 0)),
            pl.BlockSpec((d2, HID), lambda i: (0, 0)),
        ],
        out_specs=[
            pl.BlockSpec((blk, HID), lambda i: (i, 0)),
            pl.BlockSpec((blk, HID), lambda i: (i, 0)),
        ],
        out_shape=[
            jax.ShapeDtypeStruct((N, HID), jnp.float32),
            jax.ShapeDtypeStruct((N, HID), jnp.float32),
        ],
    )(feat1, feat2, e1w1, e2w1)


def _encode(a, u, w2, hn_ref, yin_ref, row):
    h = jnp.dot(a, u, preferred_element_type=jnp.float32)
    h = jnp.maximum(h, 0.0)
    norm = jnp.sqrt(jnp.sum(h * h, axis=1, keepdims=True))
    hn_ref[pl.ds(row, BLK), :] = h / jnp.maximum(norm, 1e-12)
    yin_ref[pl.ds(row, BLK), :] = jnp.dot(
        h, w2, preferred_element_type=jnp.float32)


def _simagg(hn_ref, yin_ref, row):
    hnb = hn_ref[pl.ds(row, BLK), :]
    s = jax.lax.dot_general(
        hnb, hn_ref[...],
        dimension_numbers=(((1,), (1,)), ((), ())),
        preferred_element_type=jnp.float32)
    s = jnp.where(s < THRESH, 0.0, s)
    rs = jnp.sum(s, axis=1, keepdims=True)
    agg = jnp.dot(s, yin_ref[...], preferred_element_type=jnp.float32)
    return agg / jnp.maximum(rs, 1e-12)


def _mega_kernel(a1_ref, a2_ref, u1_ref, u2_ref, w12_ref, w22_ref,
                 dw1_ref, dw2_ref,
                 y1_ref, y2_ref, z_ref, r1_ref, r2_ref,
                 hn1_ref, yin1_ref, hn2_ref, yin2_ref,
                 x1_ref, x2_ref, y1s_ref):
    i = pl.program_id(0)

    @pl.when(i < NP)
    def _p0():
        _encode(a1_ref[...], u1_ref[...], w12_ref[...],
                hn1_ref, yin1_ref, i * BLK)

    @pl.when(jnp.logical_and(i >= NP, i < 2 * NP))
    def _p1():
        j = i - NP
        _encode(a2_ref[...], u2_ref[...], w22_ref[...],
                hn2_ref, yin2_ref, j * BLK)
        y = _simagg(hn1_ref, yin1_ref, j * BLK)
        y1_ref[...] = y
        y1s_ref[pl.ds(j * BLK, BLK), :] = y
        x1_ref[pl.ds(j * BLK, BLK), :] = jnp.dot(
            y, dw1_ref[...], preferred_element_type=jnp.float32
        ).astype(jnp.bfloat16)

    @pl.when(jnp.logical_and(i >= 2 * NP, i < 3 * NP))
    def _p2():
        k = i - 2 * NP
        y = _simagg(hn2_ref, yin2_ref, k * BLK)
        y2_ref[...] = y
        z_ref[...] = (y + y1s_ref[pl.ds(k * BLK, BLK), :]) * 0.5
        x2_ref[pl.ds(k * BLK, BLK), :] = jnp.dot(
            y, dw2_ref[...], preferred_element_type=jnp.float32)
        r1_ref[...] = jnp.dot(
            a1_ref[...].astype(jnp.bfloat16), x1_ref[...],
            preferred_element_type=jnp.float32)

    @pl.when(i >= 3 * NP)
    def _p3():
        r2_ref[...] = jnp.dot(a2_ref[...], x2_ref[...],
                              preferred_element_type=jnp.float32)


def _mega(adj1, adj2, u1, u2, e1w2, e2w2, d1w, d2w):
    d1 = d1w.shape[1]
    d2 = d2w.shape[1]
    return pl.pallas_call(
        _mega_kernel,
        grid=(4 * NP,),
        in_specs=[
            pl.BlockSpec(
                (BLK, N),
                lambda i: (jnp.where(i < NP, i,
                                     jnp.clip(i - 2 * NP, 0, NP - 1)), 0)),
            pl.BlockSpec(
                (BLK, N),
                lambda i: (jnp.where(i < 3 * NP,
                                     jnp.clip(i - NP, 0, NP - 1),
                                     i - 3 * NP), 0)),
            pl.BlockSpec((N, HID), lambda i: (0, 0)),
            pl.BlockSpec((N, HID), lambda i: (0, 0)),
            pl.BlockSpec((HID, O), lambda i: (0, 0)),
            pl.BlockSpec((HID, O), lambda i: (0, 0)),
            pl.BlockSpec((O, d1), lambda i: (0, 0)),
            pl.BlockSpec((O, d2), lambda i: (0, 0)),
        ],
        out_specs=[
            pl.BlockSpec((BLK, O), lambda i: (jnp.clip(i - NP, 0, NP - 1), 0)),
            pl.BlockSpec((BLK, O),
                         lambda i: (jnp.clip(i - 2 * NP, 0, NP - 1), 0)),
            pl.BlockSpec((BLK, O),
                         lambda i: (jnp.clip(i - 2 * NP, 0, NP - 1), 0)),
            pl.BlockSpec((BLK, d1),
                         lambda i: (jnp.clip(i - 2 * NP, 0, NP - 1), 0)),
            pl.BlockSpec((BLK, d2),
                         lambda i: (jnp.clip(i - 3 * NP, 0, NP - 1), 0)),
        ],
        out_shape=[
            jax.ShapeDtypeStruct((N, O), jnp.float32),
            jax.ShapeDtypeStruct((N, O), jnp.float32),
            jax.ShapeDtypeStruct((N, O), jnp.float32),
            jax.ShapeDtypeStruct((N, d1), jnp.float32),
            jax.ShapeDtypeStruct((N, d2), jnp.float32),
        ],
        scratch_shapes=[
            pltpu.VMEM((N, HID), jnp.float32),
            pltpu.VMEM((N, O), jnp.float32),
            pltpu.VMEM((N, HID), jnp.float32),
            pltpu.VMEM((N, O), jnp.float32),
            pltpu.VMEM((N, d1), jnp.bfloat16),
            pltpu.VMEM((N, d2), jnp.float32),
            pltpu.VMEM((N, O), jnp.float32),
        ],
    )(adj1, adj2, u1, u2, e1w2, e2w2, d1w, d2w)


def kernel(feat1, feat2, adj_spatial1, adj_spatial2,
           e1w1, e1w2, e2w1, e2w2, d1w, d2w):
    u1, u2 = _u(feat1, feat2, e1w1, e2w1)
    y1, y2, z, recon1, recon2 = _mega(adj_spatial1, adj_spatial2, u1, u2,
                                      e1w2, e2w2, d1w, d2w)
    return (y1, y2, z, recon1, recon2)


# speedup vs baseline: 1.1929x; 1.0015x over previous
import jax
import jax.numpy as jnp
from jax.experimental import pallas as pl
from jax.experimental.pallas import tpu as pltpu

N = 4096
HID = 64
O = 128
THRESH = 0.6
BLK = 256
NP = N // BLK


def _u_kernel(f1_ref, f2_ref, w11_ref, w21_ref, u1_ref, u2_ref):
    u1_ref[...] = jnp.dot(f1_ref[...], w11_ref[...],
                          preferred_element_type=jnp.float32,
                  precision=jax.lax.Precision.DEFAULT)
    u2_ref[...] = jnp.dot(f2_ref[...], w21_ref[...],
                          preferred_element_type=jnp.float32,
                  precision=jax.lax.Precision.DEFAULT)


def _u(feat1, feat2, e1w1, e2w1):
    d1 = feat1.shape[1]
    d2 = feat2.shape[1]
    blk = 512
    return pl.pallas_call(
        _u_kernel,
        grid=(N // blk,),
        in_specs=[
            pl.BlockSpec((blk, d1), lambda i: (i, 0)),
            pl.BlockSpec((blk, d2), lambda i: (i, 0)),
            pl.BlockSpec((d1, HID), lambda i: (0, 0)),
            pl.BlockSpec((d2, HID), lambda i: (0, 0)),
        ],
        out_specs=[
            pl.BlockSpec((blk, HID), lambda i: (i, 0)),
            pl.BlockSpec((blk, HID), lambda i: (i, 0)),
        ],
        out_shape=[
            jax.ShapeDtypeStruct((N, HID), jnp.float32),
            jax.ShapeDtypeStruct((N, HID), jnp.float32),
        ],
    )(feat1, feat2, e1w1, e2w1)


def _encode(a, u, w2, hn_ref, yin_ref, row):
    h = jnp.dot(a, u, preferred_element_type=jnp.float32,
                  precision=jax.lax.Precision.DEFAULT)
    h = jnp.maximum(h, 0.0)
    norm = jnp.sqrt(jnp.sum(h * h, axis=1, keepdims=True))
    hn_ref[pl.ds(row, BLK), :] = h / jnp.maximum(norm, 1e-12)
    yin_ref[pl.ds(row, BLK), :] = jnp.dot(
        h, w2, preferred_element_type=jnp.float32,
                  precision=jax.lax.Precision.DEFAULT)


def _simagg(hn_ref, yin_ref, row):
    hnb = hn_ref[pl.ds(row, BLK), :]
    s = jax.lax.dot_general(
        hnb, hn_ref[...],
        dimension_numbers=(((1,), (1,)), ((), ())),
        preferred_element_type=jnp.float32,
                  precision=jax.lax.Precision.DEFAULT)
    s = jnp.where(s < THRESH, 0.0, s)
    rs = jnp.sum(s, axis=1, keepdims=True)
    agg = jnp.dot(s, yin_ref[...], preferred_element_type=jnp.float32,
                  precision=jax.lax.Precision.DEFAULT)
    return agg / jnp.maximum(rs, 1e-12)


def _mega_kernel(a1_ref, a2_ref, u1_ref, u2_ref, w12_ref, w22_ref,
                 dw1_ref, dw2_ref,
                 y1_ref, y2_ref, z_ref, r1_ref, r2_ref,
                 hn1_ref, yin1_ref, hn2_ref, yin2_ref,
                 x1_ref, x2_ref, y1s_ref):
    i = pl.program_id(0)

    @pl.when(i < NP)
    def _p0():
        _encode(a1_ref[...], u1_ref[...], w12_ref[...],
                hn1_ref, yin1_ref, i * BLK)

    @pl.when(jnp.logical_and(i >= NP, i < 2 * NP))
    def _p1():
        j = i - NP
        _encode(a2_ref[...], u2_ref[...], w22_ref[...],
                hn2_ref, yin2_ref, j * BLK)
        y = _simagg(hn1_ref, yin1_ref, j * BLK)
        y1_ref[...] = y
        y1s_ref[pl.ds(j * BLK, BLK), :] = y
        x1_ref[pl.ds(j * BLK, BLK), :] = jnp.dot(
            y, dw1_ref[...], preferred_element_type=jnp.float32,
            precision=jax.lax.Precision.DEFAULT).astype(jnp.bfloat16)

    @pl.when(jnp.logical_and(i >= 2 * NP, i < 3 * NP))
    def _p2():
        k = i - 2 * NP
        y = _simagg(hn2_ref, yin2_ref, k * BLK)
        y2_ref[...] = y
        z_ref[...] = (y + y1s_ref[pl.ds(k * BLK, BLK), :]) * 0.5
        x2_ref[pl.ds(k * BLK, BLK), :] = jnp.dot(
            y, dw2_ref[...], preferred_element_type=jnp.float32,
                  precision=jax.lax.Precision.DEFAULT)
        r1_ref[...] = jnp.dot(
            a1_ref[...].astype(jnp.bfloat16), x1_ref[...],
            preferred_element_type=jnp.float32,
                  precision=jax.lax.Precision.DEFAULT)

    @pl.when(i >= 3 * NP)
    def _p3():
        r2_ref[...] = jnp.dot(a2_ref[...], x2_ref[...],
                              preferred_element_type=jnp.float32,
                  precision=jax.lax.Precision.DEFAULT)


def _mega(adj1, adj2, u1, u2, e1w2, e2w2, d1w, d2w):
    d1 = d1w.shape[1]
    d2 = d2w.shape[1]
    return pl.pallas_call(
        _mega_kernel,
        grid=(4 * NP,),
        in_specs=[
            pl.BlockSpec(
                (BLK, N),
                lambda i: (jnp.where(i < NP, i,
                                     jnp.clip(i - 2 * NP, 0, NP - 1)), 0)),
            pl.BlockSpec(
                (BLK, N),
                lambda i: (jnp.where(i < 3 * NP,
                                     jnp.clip(i - NP, 0, NP - 1),
                                     i - 3 * NP), 0)),
            pl.BlockSpec((N, HID), lambda i: (0, 0)),
            pl.BlockSpec((N, HID), lambda i: (0, 0)),
            pl.BlockSpec((HID, O), lambda i: (0, 0)),
            pl.BlockSpec((HID, O), lambda i: (0, 0)),
            pl.BlockSpec((O, d1), lambda i: (0, 0)),
            pl.BlockSpec((O, d2), lambda i: (0, 0)),
        ],
        out_specs=[
            pl.BlockSpec((BLK, O), lambda i: (jnp.clip(i - NP, 0, NP - 1), 0)),
            pl.BlockSpec((BLK, O),
                         lambda i: (jnp.clip(i - 2 * NP, 0, NP - 1), 0)),
            pl.BlockSpec((BLK, O),
                         lambda i: (jnp.clip(i - 2 * NP, 0, NP - 1), 0)),
            pl.BlockSpec((BLK, d1),
                         lambda i: (jnp.clip(i - 2 * NP, 0, NP - 1), 0)),
            pl.BlockSpec((BLK, d2),
                         lambda i: (jnp.clip(i - 3 * NP, 0, NP - 1), 0)),
        ],
        out_shape=[
            jax.ShapeDtypeStruct((N, O), jnp.float32),
            jax.ShapeDtypeStruct((N, O), jnp.float32),
            jax.ShapeDtypeStruct((N, O), jnp.float32),
            jax.ShapeDtypeStruct((N, d1), jnp.float32),
            jax.ShapeDtypeStruct((N, d2), jnp.float32),
        ],
        scratch_shapes=[
            pltpu.VMEM((N, HID), jnp.float32),
            pltpu.VMEM((N, O), jnp.float32),
            pltpu.VMEM((N, HID), jnp.float32),
            pltpu.VMEM((N, O), jnp.float32),
            pltpu.VMEM((N, d1), jnp.bfloat16),
            pltpu.VMEM((N, d2), jnp.float32),
            pltpu.VMEM((N, O), jnp.float32),
        ],
    )(adj1, adj2, u1, u2, e1w2, e2w2, d1w, d2w)


def kernel(feat1, feat2, adj_spatial1, adj_spatial2,
           e1w1, e1w2, e2w1, e2w2, d1w, d2w):
    u1, u2 = _u(feat1, feat2, e1w1, e2w1)
    y1, y2, z, recon1, recon2 = _mega(adj_spatial1, adj_spatial2, u1, u2,
                                      e1w2, e2w2, d1w, d2w)
    return (y1, y2, z, recon1, recon2)
